# Initial kernel scaffold; baseline (speedup 1.0000x reference)
import jax
import jax.numpy as jnp
from jax.experimental import pallas as pl


def kernel(z_e, embeddings):
    raise NotImplementedError("write your pallas kernel here")


# speedup vs baseline: 1.3908x; 1.3908x over previous
import functools

import jax
import jax.numpy as jnp
from jax.experimental import pallas as pl
from jax.experimental.pallas import tpu as pltpu


def _vq_body(z_ref, et_ref, e_ref, idx_ref, zq_ref, loss_ref, *, tn, tk, k, d,
             nsteps):
    kc = k // tk
    runmin = jnp.full((tn, tk), jnp.inf, jnp.float32)
    runarg = jnp.zeros((tn, tk), jnp.int32)
    for c in range(kc):
        k0 = c * tk

        def term(dd):
            zc = z_ref[:, dd:dd + 1]
            ec = et_ref[dd:dd + 1, k0:k0 + tk]
            df = zc - ec
            return df * df

        gs = [((term(s) + term(s + 8)) + term(s + 16)) + term(s + 24)
              for s in range(8)]
        hs = [gs[s] + gs[s + 4] for s in range(4)]
        ms = [hs[s] + hs[s + 2] for s in range(2)]
        dist = ms[0] + ms[1]

        kidx = jax.lax.broadcasted_iota(jnp.int32, (tn, tk), 1) + k0
        upd = dist < runmin
        runmin = jnp.where(upd, dist, runmin)
        runarg = jnp.where(upd, kidx, runarg)

    minval = jnp.min(runmin, axis=1, keepdims=True)
    best = jnp.min(jnp.where(runmin == minval, runarg,
                             jnp.int32(2**31 - 1)), axis=1)
    idx_ref[...] = best

    kfull = jax.lax.broadcasted_iota(jnp.int32, (tn, k), 1)
    onehot = (kfull == best[:, None]).astype(jnp.float32)
    zq = jax.lax.dot_general(onehot, e_ref[...], (((1,), (0,)), ((), ())),
                             preferred_element_type=jnp.float32,
                             precision=jax.lax.Precision.HIGHEST)
    z = z_ref[...]
    zq_ref[...] = z + (zq - z)

    part = jnp.sum(minval).reshape(1, 1)

    @pl.when(pl.program_id(0) == 0)
    def _():
        loss_ref[...] = jnp.zeros((1, 1), jnp.float32)

    acc = loss_ref[...] + part

    @pl.when(pl.program_id(0) == nsteps - 1)
    def _():
        loss_ref[...] = acc * jnp.float32(1.25 / (nsteps * tn * d))

    @pl.when(pl.program_id(0) != nsteps - 1)
    def _():
        loss_ref[...] = acc


def kernel(z_e, embeddings):
    n, d = z_e.shape
    k = embeddings.shape[0]
    tn = min(128, n)
    tk = min(2048, k)
    nsteps = n // tn
    et = embeddings.T

    body = functools.partial(_vq_body, tn=tn, tk=tk, k=k, d=d, nsteps=nsteps)
    idx, zq_st, loss = pl.pallas_call(
        body,
        grid=(nsteps,),
        in_specs=[
            pl.BlockSpec((tn, d), lambda i: (i, 0)),
            pl.BlockSpec((d, k), lambda i: (0, 0)),
            pl.BlockSpec((k, d), lambda i: (0, 0)),
        ],
        out_specs=[
            pl.BlockSpec((tn,), lambda i: (i,)),
            pl.BlockSpec((tn, d), lambda i: (i, 0)),
            pl.BlockSpec((1, 1), lambda i: (0, 0)),
        ],
        out_shape=[
            jax.ShapeDtypeStruct((n,), jnp.int32),
            jax.ShapeDtypeStruct((n, d), jnp.float32),
            jax.ShapeDtypeStruct((1, 1), jnp.float32),
        ],
        compiler_params=pltpu.CompilerParams(
            dimension_semantics=("arbitrary",)),
    )(z_e, et, embeddings)
    return (zq_st, loss[0, 0], idx)
